# speedup vs baseline: 1.2360x; 1.2360x over previous
import functools

import jax
import jax.numpy as jnp
from jax.experimental import pallas as pl
from jax.experimental.pallas import tpu as pltpu

T = 2048
D = 1024
E = 8
F = 512
EP = 128
TM = 256
NT = T // TM


def _moe_kernel(x_ref, xbf_ref, wgp_ref, wg_ref, wu_ref, wd_ref,
                out_ref, comb_ref):
    e = pl.program_id(0)
    t = pl.program_id(1)

    @pl.when(jnp.logical_and(e == 0, t == 0))
    def _router():
        logits = jnp.dot(x_ref[...], wgp_ref[...],
                         preferred_element_type=jnp.float32)
        lane = jax.lax.broadcasted_iota(jnp.int32, (T, EP), 1)
        neg = jnp.float32(-1e30)
        logits = jnp.where(lane < E, logits, neg)
        m = jnp.max(logits, axis=1, keepdims=True)
        ex = jnp.exp(logits - m)
        ex = jnp.where(lane < E, ex, 0.0)
        probs = ex / jnp.sum(ex, axis=1, keepdims=True)
        m1 = jnp.max(probs, axis=1, keepdims=True)
        i1 = jnp.min(jnp.where(probs == m1, lane, EP), axis=1, keepdims=True)
        p2 = jnp.where(lane == i1, -1.0, probs)
        m2 = jnp.max(p2, axis=1, keepdims=True)
        i2 = jnp.min(jnp.where(p2 == m2, lane, EP), axis=1, keepdims=True)
        s = m1 + m2
        comb = jnp.where(lane == i1, m1 / s, 0.0) + \
               jnp.where(lane == i2, m2 / s, 0.0) + \
               jnp.where(lane == E, 1.0, 0.0)
        comb_ref[...] = comb

    rows = pl.ds(t * TM, TM)
    xb = xbf_ref[rows, :]
    g = jnp.dot(xb, wg_ref[0], preferred_element_type=jnp.float32)
    u = jnp.dot(xb, wu_ref[0], preferred_element_type=jnp.float32)
    sig = 1.0 / (1.0 + jnp.exp(-g))
    h = jnp.dot((g * sig * u).astype(jnp.bfloat16), wd_ref[0],
                preferred_element_type=jnp.float32)
    lane = jax.lax.broadcasted_iota(jnp.int32, (TM, EP), 1)
    c = comb_ref[rows, :]
    col = jnp.sum(jnp.where(lane == e, c, 0.0), axis=1, keepdims=True)
    contrib = h * col

    @pl.when(e == 0)
    def _init():
        out_ref[rows, :] = contrib

    @pl.when(e != 0)
    def _acc():
        out_ref[rows, :] = out_ref[rows, :] + contrib


def kernel(hidden_states, Wg, We_gate, We_up, We_down, Ws_gate, Ws_up, Ws_down):
    B, L, Dm = hidden_states.shape
    x = hidden_states.reshape(T, D)
    xbf = x.astype(jnp.bfloat16)
    wgp = jnp.zeros((D, EP), jnp.float32).at[:, :E].set(Wg)
    wg_all = jnp.concatenate([We_gate, Ws_gate[None]], 0).astype(jnp.bfloat16)
    wu_all = jnp.concatenate([We_up, Ws_up[None]], 0).astype(jnp.bfloat16)
    wd_all = jnp.concatenate([We_down, Ws_down[None]], 0).astype(jnp.bfloat16)

    out = pl.pallas_call(
        _moe_kernel,
        grid=(E + 1, NT),
        in_specs=[
            pl.BlockSpec((T, D), lambda e, t: (0, 0)),
            pl.BlockSpec((T, D), lambda e, t: (0, 0)),
            pl.BlockSpec((D, EP), lambda e, t: (0, 0)),
            pl.BlockSpec((1, D, F), lambda e, t: (e, 0, 0)),
            pl.BlockSpec((1, D, F), lambda e, t: (e, 0, 0)),
            pl.BlockSpec((1, F, D), lambda e, t: (e, 0, 0)),
        ],
        out_specs=pl.BlockSpec((T, D), lambda e, t: (0, 0)),
        out_shape=jax.ShapeDtypeStruct((T, D), jnp.float32),
        scratch_shapes=[pltpu.VMEM((T, EP), jnp.float32)],
    )(x, xbf, wgp, wg_all, wu_all, wd_all)
    return out.reshape(B, L, Dm)
